# baseline (speedup 1.0000x reference)
import functools

import jax
import jax.numpy as jnp
from jax import lax
from jax.experimental import pallas as pl
from jax.experimental.pallas import tpu as pltpu
from jax.experimental.pallas import tpu_sc as plsc

_N_E = 1024
_E_DIM = 256
_BETA = 0.25
_B = 9216

_BB = 512
_NB = _B // _BB


def _dist_body(z_ref, w_ref, z2_ref, w2_ref, idx_ref, loss_ref):
    s = lax.dot_general(z_ref[...], w_ref[...], (((1,), (1,)), ((), ())))
    d2 = (z2_ref[...] + w2_ref[...]) - 2.0 * s
    dists = jnp.sqrt(jnp.maximum(d2, 0.0))
    minval = jnp.min(dists, axis=1, keepdims=True)
    col = lax.broadcasted_iota(jnp.int32, dists.shape, 1)
    idx = jnp.min(jnp.where(dists == minval, col, _N_E), axis=1)
    idx_ref[0, 0, :] = idx
    @pl.when(pl.program_id(0) == 0)
    def _init():
        loss_ref[...] = jnp.zeros_like(loss_ref)
    loss_ref[...] += (1.0 + _BETA) * jnp.sum(minval * minval, keepdims=True)


_NC = 2
_NS = 16
_NW = _NC * _NS
_BPW = _B // _NW


@functools.lru_cache(maxsize=1)
def _make_sc_gather():
    mesh = plsc.VectorSubcoreMesh(core_axis_name="c", subcore_axis_name="s")

    @functools.partial(
        pl.kernel,
        mesh=mesh,
        out_type=jax.ShapeDtypeStruct((_B, _E_DIM), jnp.float32),
        scratch_types=[
            pltpu.VMEM((_BPW,), jnp.int32),
            pltpu.VMEM((_BPW, _E_DIM), jnp.float32),
            pltpu.SemaphoreType.DMA,
        ],
    )
    def _sc_gather(w_hbm, idx_hbm, out_hbm, idx_v, rows_v, sem):
        wid = lax.axis_index("s") * _NC + lax.axis_index("c")
        base = wid * _BPW
        pltpu.sync_copy(idx_hbm.at[pl.ds(base, _BPW)], idx_v)
        pltpu.async_copy(w_hbm.at[idx_v], rows_v, sem).wait()
        pltpu.sync_copy(rows_v, out_hbm.at[pl.ds(base, _BPW)])

    return _sc_gather


def _dist_call(z, W, z2, w2):
    return pl.pallas_call(
        _dist_body,
        grid=(_NB,),
        in_specs=[
            pl.BlockSpec((_BB, _E_DIM), lambda i: (i, 0)),
            pl.BlockSpec((_N_E, _E_DIM), lambda i: (0, 0)),
            pl.BlockSpec((_BB, 1), lambda i: (i, 0)),
            pl.BlockSpec((1, _N_E), lambda i: (0, 0)),
        ],
        out_specs=[
            pl.BlockSpec((1, 1, _BB), lambda i: (i, 0, 0)),
            pl.BlockSpec((1, 1), lambda i: (0, 0)),
        ],
        out_shape=[
            jax.ShapeDtypeStruct((_NB, 1, _BB), jnp.int32),
            jax.ShapeDtypeStruct((1, 1), jnp.float32),
        ],
    )(z, W, z2, w2)


def kernel(z, W):
    z2 = jnp.sum(z * z, axis=1, keepdims=True)
    w2 = jnp.sum(W * W, axis=1)[None, :]
    idx3, loss = _dist_call(z, W, z2, w2)
    idx = idx3.reshape(_B)
    z_q = _make_sc_gather()(W, idx)
    return (loss[0, 0], z_q)
